# Initial kernel scaffold; baseline (speedup 1.0000x reference)
import jax
import jax.numpy as jnp
from jax.experimental import pallas as pl


def kernel(x, edge_index, W1, b1, W2, b2):
    raise NotImplementedError("write your pallas kernel here")


# speedup vs baseline: 43.2516x; 43.2516x over previous
import functools

import jax
import jax.numpy as jnp
from jax import lax
from jax.experimental import pallas as pl
from jax.experimental.pallas import tpu as pltpu
from jax.experimental.pallas import tpu_sc as plsc

N = 10000
D_IN = 128
H = 16

NC = 2
NS = 16
NW = NC * NS
N_PAD = 10240
NPT = N_PAD // NS
E_PAD = 327680
ET = E_PAD // NW
IB = 128
CHUNK = 1024
SUB = CHUNK // IB
NCHUNK = ET // CHUNK
IRT = ET // IB
TBLK = 512


def _sc_mesh():
    return plsc.VectorSubcoreMesh(
        core_axis_name="c", subcore_axis_name="s",
        num_cores=NC, num_subcores=NS)


_SC_PARAMS = pltpu.CompilerParams(use_tc_tiling_on_sc=False)


def _zero_fill(ref, nwords):
    zeros16 = jnp.zeros((16,), jnp.float32)

    def body(i, carry):
        ref[pl.ds(i * 16, 16)] = zeros16
        return carry

    lax.fori_loop(0, nwords // 16, body, 0)


@functools.partial(
    pl.kernel,
    out_type=jax.ShapeDtypeStruct((NC, N_PAD), jnp.float32),
    mesh=_sc_mesh(),
    compiler_params=_SC_PARAMS,
    scratch_types=[
        pltpu.VMEM((SUB, IB), jnp.int32),
        pltpu.VMEM((IB,), jnp.float32),
        pltpu.VMEM((NPT,), jnp.float32),
        pltpu.VMEM_SHARED((N_PAD,), jnp.float32),
        pltpu.SemaphoreType.DMA,
    ],
)
def _sc_degree(dst_hbm, out_hbm, idx_v, ones_v, zbuf_v, acc_sh, sem):
    del sem
    c = lax.axis_index("c")
    s = lax.axis_index("s")
    tile = c * NS + s

    _zero_fill(zbuf_v, NPT)
    ones16 = jnp.ones((16,), jnp.float32)

    def ones_body(i, carry):
        ones_v[pl.ds(i * 16, 16)] = ones16
        return carry

    lax.fori_loop(0, IB // 16, ones_body, 0)
    pltpu.sync_copy(zbuf_v, acc_sh.at[pl.ds(s * NPT, NPT)])
    plsc.subcore_barrier()

    def chunk(k, carry):
        r = tile * IRT + k * SUB
        pltpu.sync_copy(dst_hbm.at[pl.ds(r, SUB)], idx_v)
        for j in range(SUB):
            pltpu.sync_copy(ones_v, acc_sh.at[idx_v.at[j]], add=True)
        return carry

    lax.fori_loop(0, NCHUNK, chunk, 0)
    plsc.subcore_barrier()
    pltpu.sync_copy(acc_sh.at[pl.ds(s * NPT, NPT)],
                    out_hbm.at[c, pl.ds(s * NPT, NPT)])


@functools.partial(
    pl.kernel,
    out_type=jax.ShapeDtypeStruct((NC, N_PAD, H), jnp.float32),
    mesh=_sc_mesh(),
    compiler_params=_SC_PARAMS,
    scratch_types=[
        pltpu.VMEM((SUB, IB), jnp.int32),
        pltpu.VMEM((SUB, IB), jnp.int32),
        pltpu.VMEM((CHUNK, H), jnp.float32),
        pltpu.VMEM((NPT, H), jnp.float32),
        pltpu.VMEM_SHARED((N_PAD, H), jnp.float32),
        pltpu.SemaphoreType.DMA,
    ],
)
def _sc_rows(h0_hbm, src_hbm, dst_hbm, out_hbm,
             src_v, dst_v, rows_v, zbuf_v, acc_sh, sem):
    c = lax.axis_index("c")
    s = lax.axis_index("s")
    tile = c * NS + s

    zeros16 = jnp.zeros((16,), jnp.float32)

    def zb(i, carry):
        zbuf_v[i] = zeros16
        return carry

    lax.fori_loop(0, NPT, zb, 0)
    pltpu.sync_copy(zbuf_v, acc_sh.at[pl.ds(s * NPT, NPT)])
    plsc.subcore_barrier()

    def chunk(k, carry):
        r = tile * IRT + k * SUB
        pltpu.sync_copy(src_hbm.at[pl.ds(r, SUB)], src_v)
        pltpu.sync_copy(dst_hbm.at[pl.ds(r, SUB)], dst_v)
        descs = [
            pltpu.async_copy(h0_hbm.at[src_v.at[j]],
                             rows_v.at[pl.ds(j * IB, IB)], sem)
            for j in range(SUB)
        ]
        for d in descs:
            d.wait()
        for j in range(SUB):
            pltpu.sync_copy(rows_v.at[pl.ds(j * IB, IB)],
                            acc_sh.at[dst_v.at[j]], add=True)
        return carry

    lax.fori_loop(0, NCHUNK, chunk, 0)
    plsc.subcore_barrier()
    pltpu.sync_copy(acc_sh.at[pl.ds(s * NPT, NPT)],
                    out_hbm.at[c, pl.ds(s * NPT, NPT)])


@functools.partial(
    pl.kernel,
    out_type=jax.ShapeDtypeStruct((NC, N_PAD), jnp.float32),
    mesh=_sc_mesh(),
    compiler_params=_SC_PARAMS,
    scratch_types=[
        pltpu.VMEM((SUB, IB), jnp.int32),
        pltpu.VMEM((SUB, IB), jnp.int32),
        pltpu.VMEM((CHUNK,), jnp.float32),
        pltpu.VMEM((NPT,), jnp.float32),
        pltpu.VMEM_SHARED((N_PAD,), jnp.float32),
        pltpu.SemaphoreType.DMA,
    ],
)
def _sc_scalar(sp_hbm, src_hbm, dst_hbm, out_hbm,
               src_v, dst_v, vals_v, zbuf_v, acc_sh, sem):
    c = lax.axis_index("c")
    s = lax.axis_index("s")
    tile = c * NS + s

    _zero_fill(zbuf_v, NPT)
    pltpu.sync_copy(zbuf_v, acc_sh.at[pl.ds(s * NPT, NPT)])
    plsc.subcore_barrier()

    def chunk(k, carry):
        r = tile * IRT + k * SUB
        pltpu.sync_copy(src_hbm.at[pl.ds(r, SUB)], src_v)
        pltpu.sync_copy(dst_hbm.at[pl.ds(r, SUB)], dst_v)
        descs = [
            pltpu.async_copy(sp_hbm.at[src_v.at[j]],
                             vals_v.at[pl.ds(j * IB, IB)], sem)
            for j in range(SUB)
        ]
        for d in descs:
            d.wait()
        for j in range(SUB):
            pltpu.sync_copy(vals_v.at[pl.ds(j * IB, IB)],
                            acc_sh.at[dst_v.at[j]], add=True)
        return carry

    lax.fori_loop(0, NCHUNK, chunk, 0)
    plsc.subcore_barrier()
    pltpu.sync_copy(acc_sh.at[pl.ds(s * NPT, NPT)],
                    out_hbm.at[c, pl.ds(s * NPT, NPT)])


def _tc1_body(x_ref, w1_ref, indeg_ref, h0p_ref, dinv_ref):
    ind = indeg_ref[...]
    deg = ind[:, 0:1] + ind[:, 1:2] + 1.0
    dinv = lax.rsqrt(deg)
    h0 = jnp.dot(x_ref[...], w1_ref[...], preferred_element_type=jnp.float32)
    h0p_ref[...] = h0 * dinv
    dinv_ref[...] = dinv


_tc1 = pl.pallas_call(
    _tc1_body,
    grid=(N_PAD // TBLK,),
    in_specs=[
        pl.BlockSpec((TBLK, D_IN), lambda i: (i, 0)),
        pl.BlockSpec((D_IN, H), lambda i: (0, 0)),
        pl.BlockSpec((TBLK, 2), lambda i: (i, 0)),
    ],
    out_specs=[
        pl.BlockSpec((TBLK, H), lambda i: (i, 0)),
        pl.BlockSpec((TBLK, 1), lambda i: (i, 0)),
    ],
    out_shape=[
        jax.ShapeDtypeStruct((N_PAD, H), jnp.float32),
        jax.ShapeDtypeStruct((N_PAD, 1), jnp.float32),
    ],
)


def _tc2_body(a_ref, b_ref, h0p_ref, dinv_ref, b1_ref, w2_ref, sp_ref):
    dinv = dinv_ref[...]
    z = dinv * (a_ref[...] + b_ref[...] + h0p_ref[...])
    h = jnp.maximum(z + b1_ref[...], 0.0)
    s = jnp.dot(h, w2_ref[...], preferred_element_type=jnp.float32)
    sp_ref[...] = s * dinv


_tc2 = pl.pallas_call(
    _tc2_body,
    grid=(N_PAD // TBLK,),
    in_specs=[
        pl.BlockSpec((TBLK, H), lambda i: (i, 0)),
        pl.BlockSpec((TBLK, H), lambda i: (i, 0)),
        pl.BlockSpec((TBLK, H), lambda i: (i, 0)),
        pl.BlockSpec((TBLK, 1), lambda i: (i, 0)),
        pl.BlockSpec((1, H), lambda i: (0, 0)),
        pl.BlockSpec((H, 1), lambda i: (0, 0)),
    ],
    out_specs=pl.BlockSpec((TBLK, 1), lambda i: (i, 0)),
    out_shape=jax.ShapeDtypeStruct((N_PAD, 1), jnp.float32),
)


def _tc3_body(a_ref, b_ref, sp_ref, dinv_ref, b2_ref, out_ref):
    t = dinv_ref[...] * (a_ref[...] + b_ref[...] + sp_ref[...]) + b2_ref[...]
    out_ref[...] = jax.nn.sigmoid(t)


_tc3 = pl.pallas_call(
    _tc3_body,
    grid=(N_PAD // TBLK,),
    in_specs=[
        pl.BlockSpec((TBLK, 1), lambda i: (i, 0)),
        pl.BlockSpec((TBLK, 1), lambda i: (i, 0)),
        pl.BlockSpec((TBLK, 1), lambda i: (i, 0)),
        pl.BlockSpec((TBLK, 1), lambda i: (i, 0)),
        pl.BlockSpec((1, 1), lambda i: (0, 0)),
    ],
    out_specs=pl.BlockSpec((TBLK, 1), lambda i: (i, 0)),
    out_shape=jax.ShapeDtypeStruct((N_PAD, 1), jnp.float32),
)


def kernel(x, edge_index, W1, b1, W2, b2):
    x_pad = jnp.pad(x, ((0, N_PAD - N), (0, 0)))
    src = edge_index[0]
    dst = edge_index[1]
    n_fake = E_PAD - src.shape[0]
    pad_idx = N + (jnp.arange(n_fake, dtype=jnp.int32) % (N_PAD - N))
    src_p = jnp.concatenate([src, pad_idx]).reshape(E_PAD // IB, IB)
    dst_p = jnp.concatenate([dst, pad_idx]).reshape(E_PAD // IB, IB)

    indeg = _sc_degree(dst_p)
    h0p, dinv = _tc1(x_pad, W1, jnp.transpose(indeg))
    agg1 = _sc_rows(h0p, src_p, dst_p)
    sp = _tc2(agg1[0], agg1[1], h0p, dinv, b1.reshape(1, H), W2)
    agg2 = _sc_scalar(sp.reshape(N_PAD), src_p, dst_p)
    out = _tc3(agg2[0][:, None], agg2[1][:, None], sp, dinv,
               b2.reshape(1, 1))
    return out[:N]


# speedup vs baseline: 70.9912x; 1.6414x over previous
import functools

import jax
import jax.numpy as jnp
from jax import lax
from jax.experimental import pallas as pl
from jax.experimental.pallas import tpu as pltpu
from jax.experimental.pallas import tpu_sc as plsc

N = 10000
D_IN = 128
H = 16

NC = 2
NS = 16
NW = NC * NS
N_PAD = 10240
NPT = N_PAD // NS
E_PAD = 327680
ET = E_PAD // NW
IB = 128
CHUNK = 1024
SUB = CHUNK // IB
NCHUNK = ET // CHUNK
IRT = ET // IB
EROWS = E_PAD // IB


def _sc_mesh():
    return plsc.VectorSubcoreMesh(
        core_axis_name="c", subcore_axis_name="s",
        num_cores=NC, num_subcores=NS)


_SC_PARAMS = pltpu.CompilerParams(use_tc_tiling_on_sc=False)
_SC_PARAMS_NL = pltpu.CompilerParams(use_tc_tiling_on_sc=False,
                                     needs_layout_passes=False)


def _fill(ref, nwords, value16):
    def body(i, carry):
        ref[pl.ds(i * 16, 16)] = value16
        return carry

    lax.fori_loop(0, nwords // 16, body, 0)


@functools.partial(
    pl.kernel,
    out_type=jax.ShapeDtypeStruct((NC, N_PAD), jnp.float32),
    mesh=_sc_mesh(),
    compiler_params=_SC_PARAMS,
    scratch_types=[
        pltpu.VMEM((2, SUB, IB), jnp.int32),
        pltpu.VMEM((IB,), jnp.float32),
        pltpu.VMEM((NPT,), jnp.float32),
        pltpu.VMEM_SHARED((N_PAD,), jnp.float32),
        pltpu.SemaphoreType.DMA,
        pltpu.SemaphoreType.DMA,
    ],
)
def _sc_degree(dst_hbm, out_hbm, idx_v, ones_v, zbuf_v, acc_sh, sem0, sem1):
    c = lax.axis_index("c")
    s = lax.axis_index("s")
    tile = c * NS + s
    sems = (sem0, sem1)

    zeros16 = jnp.zeros((16,), jnp.float32)
    _fill(zbuf_v, NPT, zeros16)
    _fill(ones_v, IB, jnp.ones((16,), jnp.float32))
    pltpu.sync_copy(zbuf_v, acc_sh.at[pl.ds(s * NPT, NPT)])
    plsc.subcore_barrier()

    pending = [None, None]
    for k in range(NCHUNK):
        b = k & 1
        if pending[b] is not None:
            for d in pending[b]:
                d.wait()
        r = tile * IRT + k * SUB
        pltpu.sync_copy(dst_hbm.at[pl.ds(r, SUB)], idx_v.at[b])
        pending[b] = [
            pltpu.async_copy(ones_v, acc_sh.at[idx_v.at[b, j]], sems[b],
                             add=True)
            for j in range(SUB)
        ]
    for b in range(2):
        for d in pending[b]:
            d.wait()
    plsc.subcore_barrier()
    pltpu.sync_copy(acc_sh.at[pl.ds(s * NPT, NPT)],
                    out_hbm.at[c, pl.ds(s * NPT, NPT)])


@functools.partial(
    pl.kernel,
    out_type=jax.ShapeDtypeStruct((NC, N_PAD, H), jnp.float32),
    mesh=_sc_mesh(),
    compiler_params=_SC_PARAMS,
    scratch_types=[
        pltpu.VMEM((2, SUB, IB), jnp.int32),
        pltpu.VMEM((2, SUB, IB), jnp.int32),
        pltpu.VMEM((2, CHUNK, H), jnp.float32),
        pltpu.VMEM((NPT, H), jnp.float32),
        pltpu.VMEM_SHARED((N_PAD, H), jnp.float32),
        pltpu.SemaphoreType.DMA,
        pltpu.SemaphoreType.DMA,
        pltpu.SemaphoreType.DMA,
    ],
)
def _sc_rows(h0_hbm, src_hbm, dst_hbm, out_hbm,
             src_v, dst_v, rows_v, zbuf_v, acc_sh, gsem, ssem0, ssem1):
    c = lax.axis_index("c")
    s = lax.axis_index("s")
    tile = c * NS + s
    ssems = (ssem0, ssem1)

    zeros16 = jnp.zeros((16,), jnp.float32)

    def zb(i, carry):
        zbuf_v[i] = zeros16
        return carry

    lax.fori_loop(0, NPT, zb, 0)
    pltpu.sync_copy(zbuf_v, acc_sh.at[pl.ds(s * NPT, NPT)])
    plsc.subcore_barrier()

    pending = [None, None]
    for k in range(NCHUNK):
        b = k & 1
        if pending[b] is not None:
            for d in pending[b]:
                d.wait()
        r = tile * IRT + k * SUB
        pltpu.sync_copy(src_hbm.at[pl.ds(r, SUB)], src_v.at[b])
        pltpu.sync_copy(dst_hbm.at[pl.ds(r, SUB)], dst_v.at[b])
        gathers = [
            pltpu.async_copy(h0_hbm.at[src_v.at[b, j]],
                             rows_v.at[b, pl.ds(j * IB, IB)], gsem)
            for j in range(SUB)
        ]
        for d in gathers:
            d.wait()
        pending[b] = [
            pltpu.async_copy(rows_v.at[b, pl.ds(j * IB, IB)],
                             acc_sh.at[dst_v.at[b, j]], ssems[b], add=True)
            for j in range(SUB)
        ]
    for b in range(2):
        for d in pending[b]:
            d.wait()
    plsc.subcore_barrier()
    pltpu.sync_copy(acc_sh.at[pl.ds(s * NPT, NPT)],
                    out_hbm.at[c, pl.ds(s * NPT, NPT)])


@functools.partial(
    pl.kernel,
    out_type=jax.ShapeDtypeStruct((NC, N_PAD), jnp.float32),
    mesh=_sc_mesh(),
    compiler_params=_SC_PARAMS_NL,
    scratch_types=[
        pltpu.VMEM((2, CHUNK), jnp.int32),
        pltpu.VMEM((2, SUB, IB), jnp.int32),
        pltpu.VMEM((2, CHUNK), jnp.float32),
        pltpu.VMEM((N_PAD,), jnp.float32),
        pltpu.VMEM((NPT,), jnp.float32),
        pltpu.VMEM_SHARED((N_PAD,), jnp.float32),
        pltpu.SemaphoreType.DMA,
        pltpu.SemaphoreType.DMA,
    ],
)
def _sc_scalar(sp_hbm, srcf_hbm, dst_hbm, out_hbm,
               src_v, dst_v, vals_v, sp_v, zbuf_v, acc_sh, ssem0, ssem1):
    c = lax.axis_index("c")
    s = lax.axis_index("s")
    tile = c * NS + s
    ssems = (ssem0, ssem1)

    zeros16 = jnp.zeros((16,), jnp.float32)
    _fill(zbuf_v, NPT, zeros16)
    pltpu.sync_copy(zbuf_v, acc_sh.at[pl.ds(s * NPT, NPT)])
    pltpu.sync_copy(sp_hbm, sp_v)
    plsc.subcore_barrier()

    pending = [None, None]
    for k in range(NCHUNK):
        b = k & 1
        if pending[b] is not None:
            for d in pending[b]:
                d.wait()
        base = tile * ET + k * CHUNK
        r = tile * IRT + k * SUB
        pltpu.sync_copy(srcf_hbm.at[pl.ds(base, CHUNK)], src_v.at[b])
        pltpu.sync_copy(dst_hbm.at[pl.ds(r, SUB)], dst_v.at[b])

        def gat(i, carry, b=b):
            idx16 = src_v[b, pl.ds(i * 16, 16)]
            vals_v[b, pl.ds(i * 16, 16)] = plsc.load_gather(sp_v, [idx16])
            return carry

        lax.fori_loop(0, CHUNK // 16, gat, 0)
        pending[b] = [
            pltpu.async_copy(vals_v.at[b, pl.ds(j * IB, IB)],
                             acc_sh.at[dst_v.at[b, j]], ssems[b], add=True)
            for j in range(SUB)
        ]
    for b in range(2):
        for d in pending[b]:
            d.wait()
    plsc.subcore_barrier()
    pltpu.sync_copy(acc_sh.at[pl.ds(s * NPT, NPT)],
                    out_hbm.at[c, pl.ds(s * NPT, NPT)])


def _tc1_body(x_ref, w1_ref, indeg_ref, h0p_ref, dinv_ref):
    ind = indeg_ref[...]
    deg = jnp.transpose(ind[0:1, :] + ind[1:2, :] + 1.0)
    dinv = lax.rsqrt(deg)
    h0 = jnp.dot(x_ref[...], w1_ref[...], preferred_element_type=jnp.float32)
    h0p_ref[...] = h0 * dinv
    dinv_ref[...] = dinv


_tc1 = pl.pallas_call(
    _tc1_body,
    out_shape=[
        jax.ShapeDtypeStruct((N_PAD, H), jnp.float32),
        jax.ShapeDtypeStruct((N_PAD, 1), jnp.float32),
    ],
)


def _tc2_body(agg1_ref, h0p_ref, dinv_ref, b1_ref, w2_ref, sp_ref):
    dinv = dinv_ref[...]
    z = dinv * (agg1_ref[0] + agg1_ref[1] + h0p_ref[...])
    h = jnp.maximum(z + b1_ref[...], 0.0)
    s = jnp.sum(h * w2_ref[...], axis=1, keepdims=True)
    sp_ref[...] = s * dinv


_tc2 = pl.pallas_call(
    _tc2_body,
    out_shape=jax.ShapeDtypeStruct((N_PAD, 1), jnp.float32),
)


def _tc3_body(agg2_ref, sp_ref, dinv_ref, b2_ref, out_ref):
    a2 = agg2_ref[...]
    a2c = jnp.transpose(a2[0:1, :] + a2[1:2, :])
    t = dinv_ref[...] * (a2c + sp_ref[...]) + b2_ref[...]
    out_ref[...] = jax.nn.sigmoid(t)


_tc3 = pl.pallas_call(
    _tc3_body,
    out_shape=jax.ShapeDtypeStruct((N_PAD, 1), jnp.float32),
)


def kernel(x, edge_index, W1, b1, W2, b2):
    x_pad = jnp.pad(x, ((0, N_PAD - N), (0, 0)))
    n_fake = E_PAD - edge_index.shape[1]
    pad_idx = N + (jnp.arange(n_fake, dtype=jnp.int32) % (N_PAD - N))
    ei = jnp.concatenate(
        [edge_index, jnp.broadcast_to(pad_idx, (2, n_fake))], axis=1)
    src2d = ei[0].reshape(EROWS, IB)
    dst2d = ei[1].reshape(EROWS, IB)
    srcf = ei[0]

    indeg = _sc_degree(dst2d)
    h0p, dinv = _tc1(x_pad, W1, indeg)
    agg1 = _sc_rows(h0p, src2d, dst2d)
    sp = _tc2(agg1, h0p, dinv, b1.reshape(1, H), W2.reshape(1, H))
    agg2 = _sc_scalar(sp.reshape(N_PAD), srcf, dst2d)
    out = _tc3(agg2, sp, dinv, b2.reshape(1, 1))
    return out[:N]


# speedup vs baseline: 73.4299x; 1.0344x over previous
import functools

import numpy as np

import jax
import jax.numpy as jnp
from jax import lax
from jax.experimental import pallas as pl
from jax.experimental.pallas import tpu as pltpu
from jax.experimental.pallas import tpu_sc as plsc

N = 10000
D_IN = 128
H = 16

NC = 2
NS = 16
NW = NC * NS
N_PAD = 10240
NPT = N_PAD // NS
E_PAD = 327680
ET = E_PAD // NW
IB = 128
CHUNK = 1024
SUB = CHUNK // IB
NCHUNK = ET // CHUNK
IRT = ET // IB
EROWS = E_PAD // IB


def _sc_mesh():
    return plsc.VectorSubcoreMesh(
        core_axis_name="c", subcore_axis_name="s",
        num_cores=NC, num_subcores=NS)


_SC_PARAMS = pltpu.CompilerParams(use_tc_tiling_on_sc=False)
_SC_PARAMS_NL = pltpu.CompilerParams(use_tc_tiling_on_sc=False,
                                     needs_layout_passes=False)


def _fill(ref, nwords, value16):
    def body(i, carry):
        ref[pl.ds(i * 16, 16)] = value16
        return carry

    lax.fori_loop(0, nwords // 16, body, 0)


@functools.partial(
    pl.kernel,
    out_type=jax.ShapeDtypeStruct((NC, N_PAD), jnp.float32),
    mesh=_sc_mesh(),
    compiler_params=_SC_PARAMS,
    scratch_types=[
        pltpu.VMEM((2, SUB, IB), jnp.int32),
        pltpu.VMEM((IB,), jnp.float32),
        pltpu.VMEM((NPT,), jnp.float32),
        pltpu.VMEM_SHARED((N_PAD,), jnp.float32),
        pltpu.SemaphoreType.DMA,
        pltpu.SemaphoreType.DMA,
    ],
)
def _sc_degree(dst_hbm, out_hbm, idx_v, ones_v, zbuf_v, acc_sh, sem0, sem1):
    c = lax.axis_index("c")
    s = lax.axis_index("s")
    tile = c * NS + s
    sems = (sem0, sem1)

    zeros16 = jnp.zeros((16,), jnp.float32)
    _fill(zbuf_v, NPT, zeros16)
    _fill(ones_v, IB, jnp.ones((16,), jnp.float32))
    pltpu.sync_copy(zbuf_v, acc_sh.at[pl.ds(s * NPT, NPT)])
    plsc.subcore_barrier()

    pending = [None, None]
    for k in range(NCHUNK):
        b = k & 1
        if pending[b] is not None:
            for d in pending[b]:
                d.wait()
        r = tile * IRT + k * SUB
        pltpu.sync_copy(dst_hbm.at[pl.ds(r, SUB)], idx_v.at[b])
        pending[b] = [
            pltpu.async_copy(ones_v, acc_sh.at[idx_v.at[b, j]], sems[b],
                             add=True)
            for j in range(SUB)
        ]
    for b in range(2):
        for d in pending[b]:
            d.wait()
    plsc.subcore_barrier()
    pltpu.sync_copy(acc_sh.at[pl.ds(s * NPT, NPT)],
                    out_hbm.at[c, pl.ds(s * NPT, NPT)])


@functools.partial(
    pl.kernel,
    out_type=jax.ShapeDtypeStruct((NC, N_PAD, H), jnp.float32),
    mesh=_sc_mesh(),
    compiler_params=_SC_PARAMS,
    scratch_types=[
        pltpu.VMEM((2, SUB, IB), jnp.int32),
        pltpu.VMEM((2, SUB, IB), jnp.int32),
        pltpu.VMEM((2, CHUNK, H), jnp.float32),
        pltpu.VMEM((NPT, H), jnp.float32),
        pltpu.VMEM_SHARED((N_PAD, H), jnp.float32),
        pltpu.SemaphoreType.DMA,
        pltpu.SemaphoreType.DMA,
        pltpu.SemaphoreType.DMA,
    ],
)
def _sc_rows(h0_hbm, src_hbm, dst_hbm, out_hbm,
             src_v, dst_v, rows_v, zbuf_v, acc_sh, gsem, ssem0, ssem1):
    c = lax.axis_index("c")
    s = lax.axis_index("s")
    tile = c * NS + s
    ssems = (ssem0, ssem1)

    zeros16 = jnp.zeros((16,), jnp.float32)

    def zb(i, carry):
        zbuf_v[i] = zeros16
        return carry

    lax.fori_loop(0, NPT, zb, 0)
    pltpu.sync_copy(zbuf_v, acc_sh.at[pl.ds(s * NPT, NPT)])
    plsc.subcore_barrier()

    pending = [None, None]
    for k in range(NCHUNK):
        b = k & 1
        if pending[b] is not None:
            for d in pending[b]:
                d.wait()
        r = tile * IRT + k * SUB
        pltpu.sync_copy(src_hbm.at[pl.ds(r, SUB)], src_v.at[b])
        pltpu.sync_copy(dst_hbm.at[pl.ds(r, SUB)], dst_v.at[b])
        gathers = [
            pltpu.async_copy(h0_hbm.at[src_v.at[b, j]],
                             rows_v.at[b, pl.ds(j * IB, IB)], gsem)
            for j in range(SUB)
        ]
        for d in gathers:
            d.wait()
        pending[b] = [
            pltpu.async_copy(rows_v.at[b, pl.ds(j * IB, IB)],
                             acc_sh.at[dst_v.at[b, j]], ssems[b], add=True)
            for j in range(SUB)
        ]
    for b in range(2):
        for d in pending[b]:
            d.wait()
    plsc.subcore_barrier()
    pltpu.sync_copy(acc_sh.at[pl.ds(s * NPT, NPT)],
                    out_hbm.at[c, pl.ds(s * NPT, NPT)])


def _rsqrt16(d):
    i = plsc.bitcast(d, jnp.int32)
    i = jnp.int32(0x5F3759DF) - lax.shift_right_logical(i, 1)
    y = plsc.bitcast(i, jnp.float32)
    for _ in range(3):
        y = y * (1.5 - 0.5 * d * y * y)
    return y


@functools.partial(
    pl.kernel,
    out_type=[
        jax.ShapeDtypeStruct((NC, N_PAD), jnp.float32),
        jax.ShapeDtypeStruct((N_PAD,), jnp.float32),
    ],
    mesh=_sc_mesh(),
    compiler_params=_SC_PARAMS_NL,
    scratch_types=[
        pltpu.VMEM((NPT, H), jnp.float32),
        pltpu.VMEM((NPT, H), jnp.float32),
        pltpu.VMEM((NPT, H), jnp.float32),
        pltpu.VMEM((NPT,), jnp.float32),
        pltpu.VMEM((NPT,), jnp.float32),
        pltpu.VMEM((NPT,), jnp.float32),
        pltpu.VMEM((NPT,), jnp.float32),
        pltpu.VMEM((16,), jnp.float32),
        pltpu.VMEM((16,), jnp.float32),
        pltpu.VMEM((2, CHUNK), jnp.int32),
        pltpu.VMEM((2, SUB, IB), jnp.int32),
        pltpu.VMEM((2, CHUNK), jnp.float32),
        pltpu.VMEM((N_PAD,), jnp.float32),
        pltpu.VMEM_SHARED((N_PAD,), jnp.float32),
        pltpu.VMEM_SHARED((N_PAD,), jnp.float32),
        pltpu.SemaphoreType.DMA,
        pltpu.SemaphoreType.DMA,
    ],
)
def _sc_layer2(agg1_hbm, h0p_hbm, indeg_hbm, b1_hbm, w2_hbm, srcf_hbm,
               dst_hbm, out_hbm, sp_hbm,
               a0_v, a1_v, h0_v, i0_v, i1_v, dinv_v, sps_v, b1_v, w2_v,
               src_v, dst_v, vals_v, spt_v, sp_sh, acc_sh, ssem0, ssem1):
    c = lax.axis_index("c")
    s = lax.axis_index("s")
    tile = c * NS + s
    ssems = (ssem0, ssem1)
    lo = s * NPT

    zeros16 = jnp.zeros((16,), jnp.float32)
    _fill(sps_v, NPT, zeros16)
    pltpu.sync_copy(sps_v, acc_sh.at[pl.ds(lo, NPT)])

    pltpu.sync_copy(agg1_hbm.at[0, pl.ds(lo, NPT)], a0_v)
    pltpu.sync_copy(agg1_hbm.at[1, pl.ds(lo, NPT)], a1_v)
    pltpu.sync_copy(h0p_hbm.at[pl.ds(lo, NPT)], h0_v)
    pltpu.sync_copy(indeg_hbm.at[0, pl.ds(lo, NPT)], i0_v)
    pltpu.sync_copy(indeg_hbm.at[1, pl.ds(lo, NPT)], i1_v)
    pltpu.sync_copy(b1_hbm, b1_v)
    pltpu.sync_copy(w2_hbm, w2_v)

    def dv(i, carry):
        d = i0_v[pl.ds(i * 16, 16)] + i1_v[pl.ds(i * 16, 16)] + 1.0
        dinv_v[pl.ds(i * 16, 16)] = _rsqrt16(d)
        return carry

    lax.fori_loop(0, NPT // 16, dv, 0)

    b1r = b1_v[...]
    w2r = w2_v[...]
    lastlane = lax.broadcasted_iota(jnp.int32, (16,), 0) == 15

    def node(n, carry):
        idxn = jnp.full((16,), n, jnp.int32)
        dvn = plsc.load_gather(dinv_v, [idxn])
        row = a0_v[n] + a1_v[n] + h0_v[n]
        h = jnp.maximum(row * dvn + b1r, 0.0)
        cs = plsc.cumsum(h * w2r)
        plsc.store_scatter(sps_v, [idxn], cs * dvn, mask=lastlane)
        return carry

    lax.fori_loop(0, NPT, node, 0)

    pltpu.sync_copy(sps_v, sp_sh.at[pl.ds(lo, NPT)])

    @pl.when(c == 0)
    def _():
        pltpu.sync_copy(sps_v, sp_hbm.at[pl.ds(lo, NPT)])

    plsc.subcore_barrier()
    pltpu.sync_copy(sp_sh, spt_v)

    pending = [None, None]
    for k in range(NCHUNK):
        b = k & 1
        if pending[b] is not None:
            for d in pending[b]:
                d.wait()
        base = tile * ET + k * CHUNK
        r = tile * IRT + k * SUB
        pltpu.sync_copy(srcf_hbm.at[pl.ds(base, CHUNK)], src_v.at[b])
        pltpu.sync_copy(dst_hbm.at[pl.ds(r, SUB)], dst_v.at[b])

        def gat(i, carry, b=b):
            idx16 = src_v[b, pl.ds(i * 16, 16)]
            vals_v[b, pl.ds(i * 16, 16)] = plsc.load_gather(spt_v, [idx16])
            return carry

        lax.fori_loop(0, CHUNK // 16, gat, 0)
        pending[b] = [
            pltpu.async_copy(vals_v.at[b, pl.ds(j * IB, IB)],
                             acc_sh.at[dst_v.at[b, j]], ssems[b], add=True)
            for j in range(SUB)
        ]
    for b in range(2):
        for d in pending[b]:
            d.wait()
    plsc.subcore_barrier()
    pltpu.sync_copy(acc_sh.at[pl.ds(lo, NPT)],
                    out_hbm.at[c, pl.ds(lo, NPT)])


def _tca_body(x_ref, w1_ref, h0_ref):
    h0_ref[...] = jnp.dot(x_ref[...], w1_ref[...],
                          preferred_element_type=jnp.float32)


_tca = pl.pallas_call(
    _tca_body,
    out_shape=jax.ShapeDtypeStruct((N_PAD, H), jnp.float32),
)


def _tcb_body(h0_ref, indeg_ref, h0p_ref, dinv_ref):
    ind = indeg_ref[...]
    deg = jnp.transpose(ind[0:1, :] + ind[1:2, :] + 1.0)
    dinv = lax.rsqrt(deg)
    h0p_ref[...] = h0_ref[...] * dinv
    dinv_ref[...] = dinv


_tcb = pl.pallas_call(
    _tcb_body,
    out_shape=[
        jax.ShapeDtypeStruct((N_PAD, H), jnp.float32),
        jax.ShapeDtypeStruct((N_PAD, 1), jnp.float32),
    ],
)


def _tc3_body(agg2_ref, sp_ref, dinv_ref, b2_ref, out_ref):
    a2 = agg2_ref[...]
    tcol = jnp.transpose(a2[0:1, :] + a2[1:2, :] + sp_ref[...])
    t = dinv_ref[...] * tcol + b2_ref[...]
    out_ref[...] = jax.nn.sigmoid(t)


_tc3 = pl.pallas_call(
    _tc3_body,
    out_shape=jax.ShapeDtypeStruct((N_PAD, 1), jnp.float32),
)


_PAD_IDX = np.asarray(
    N + (np.arange(E_PAD - 320000, dtype=np.int32) % (N_PAD - N)),
    dtype=np.int32)


def kernel(x, edge_index, W1, b1, W2, b2):
    x_pad = jnp.pad(x, ((0, N_PAD - N), (0, 0)))
    n_fake = E_PAD - edge_index.shape[1]
    pad2 = jnp.broadcast_to(jnp.asarray(_PAD_IDX[:n_fake]), (2, n_fake))
    ei = jnp.concatenate([edge_index, pad2], axis=1)
    src2d = ei[0].reshape(EROWS, IB)
    dst2d = ei[1].reshape(EROWS, IB)
    srcf = ei[0]

    h0 = _tca(x_pad, W1)
    indeg = _sc_degree(dst2d)
    h0p, dinv = _tcb(h0, indeg)
    agg1 = _sc_rows(h0p, src2d, dst2d)
    agg2, sp = _sc_layer2(agg1, h0p, indeg, b1, W2.reshape(H), srcf, dst2d)
    out = _tc3(agg2, sp.reshape(1, N_PAD), dinv, b2.reshape(1, 1))
    return out[:N]


# speedup vs baseline: 74.7065x; 1.0174x over previous
import functools

import numpy as np

import jax
import jax.numpy as jnp
from jax import lax
from jax.experimental import pallas as pl
from jax.experimental.pallas import tpu as pltpu
from jax.experimental.pallas import tpu_sc as plsc

N = 10000
D_IN = 128
H = 16

NC = 2
NS = 16
NW = NC * NS
N_PAD = 10240
NPT = N_PAD // NS
E_PAD = 327680
ET = E_PAD // NW
IB = 128
CHUNK = 1024
SUB = CHUNK // IB
NCHUNK = ET // CHUNK
RCHUNK = 2048
RSUB = RCHUNK // IB
RNCHUNK = ET // RCHUNK
IRT = ET // IB
EROWS = E_PAD // IB


def _sc_mesh():
    return plsc.VectorSubcoreMesh(
        core_axis_name="c", subcore_axis_name="s",
        num_cores=NC, num_subcores=NS)


_SC_PARAMS = pltpu.CompilerParams(use_tc_tiling_on_sc=False)
_SC_PARAMS_NL = pltpu.CompilerParams(use_tc_tiling_on_sc=False,
                                     needs_layout_passes=False)


def _fill(ref, nwords, value16):
    def body(i, carry):
        ref[pl.ds(i * 16, 16)] = value16
        return carry

    lax.fori_loop(0, nwords // 16, body, 0)


@functools.partial(
    pl.kernel,
    out_type=jax.ShapeDtypeStruct((NC, N_PAD), jnp.float32),
    mesh=_sc_mesh(),
    compiler_params=_SC_PARAMS,
    scratch_types=[
        pltpu.VMEM((2, SUB, IB), jnp.int32),
        pltpu.VMEM((IB,), jnp.float32),
        pltpu.VMEM((NPT,), jnp.float32),
        pltpu.VMEM_SHARED((N_PAD,), jnp.float32),
        pltpu.SemaphoreType.DMA,
        pltpu.SemaphoreType.DMA,
    ],
)
def _sc_degree(dst_hbm, out_hbm, idx_v, ones_v, zbuf_v, acc_sh, sem0, sem1):
    c = lax.axis_index("c")
    s = lax.axis_index("s")
    tile = c * NS + s
    sems = (sem0, sem1)

    zeros16 = jnp.zeros((16,), jnp.float32)
    _fill(zbuf_v, NPT, zeros16)
    _fill(ones_v, IB, jnp.ones((16,), jnp.float32))
    pltpu.sync_copy(zbuf_v, acc_sh.at[pl.ds(s * NPT, NPT)])
    plsc.subcore_barrier()

    pending = [None, None]
    for k in range(NCHUNK):
        b = k & 1
        if pending[b] is not None:
            for d in pending[b]:
                d.wait()
        r = tile * IRT + k * SUB
        pltpu.sync_copy(dst_hbm.at[pl.ds(r, SUB)], idx_v.at[b])
        pending[b] = [
            pltpu.async_copy(ones_v, acc_sh.at[idx_v.at[b, j]], sems[b],
                             add=True)
            for j in range(SUB)
        ]
    for b in range(2):
        for d in pending[b]:
            d.wait()
    plsc.subcore_barrier()
    pltpu.sync_copy(acc_sh.at[pl.ds(s * NPT, NPT)],
                    out_hbm.at[c, pl.ds(s * NPT, NPT)])


@functools.partial(
    pl.kernel,
    out_type=jax.ShapeDtypeStruct((NC, N_PAD, H), jnp.float32),
    mesh=_sc_mesh(),
    compiler_params=_SC_PARAMS,
    scratch_types=[
        pltpu.VMEM((2, RSUB, IB), jnp.int32),
        pltpu.VMEM((2, RSUB, IB), jnp.int32),
        pltpu.VMEM((2, RCHUNK, H), jnp.float32),
        pltpu.VMEM((NPT, H), jnp.float32),
        pltpu.VMEM_SHARED((N_PAD, H), jnp.float32),
        pltpu.SemaphoreType.DMA,
        pltpu.SemaphoreType.DMA,
        pltpu.SemaphoreType.DMA,
    ],
)
def _sc_rows(h0_hbm, src_hbm, dst_hbm, out_hbm,
             src_v, dst_v, rows_v, zbuf_v, acc_sh, gsem, ssem0, ssem1):
    c = lax.axis_index("c")
    s = lax.axis_index("s")
    tile = c * NS + s
    ssems = (ssem0, ssem1)

    zeros16 = jnp.zeros((16,), jnp.float32)

    def zb(i, carry):
        zbuf_v[i] = zeros16
        return carry

    lax.fori_loop(0, NPT, zb, 0)
    pltpu.sync_copy(zbuf_v, acc_sh.at[pl.ds(s * NPT, NPT)])
    plsc.subcore_barrier()

    pending = [None, None]
    for k in range(RNCHUNK):
        b = k & 1
        if pending[b] is not None:
            for d in pending[b]:
                d.wait()
        r = tile * IRT + k * RSUB
        pltpu.sync_copy(src_hbm.at[pl.ds(r, RSUB)], src_v.at[b])
        pltpu.sync_copy(dst_hbm.at[pl.ds(r, RSUB)], dst_v.at[b])
        gathers = [
            pltpu.async_copy(h0_hbm.at[src_v.at[b, j]],
                             rows_v.at[b, pl.ds(j * IB, IB)], gsem)
            for j in range(RSUB)
        ]
        for d in gathers:
            d.wait()
        pending[b] = [
            pltpu.async_copy(rows_v.at[b, pl.ds(j * IB, IB)],
                             acc_sh.at[dst_v.at[b, j]], ssems[b], add=True)
            for j in range(RSUB)
        ]
    for b in range(2):
        for d in pending[b]:
            d.wait()
    plsc.subcore_barrier()
    pltpu.sync_copy(acc_sh.at[pl.ds(s * NPT, NPT)],
                    out_hbm.at[c, pl.ds(s * NPT, NPT)])


def _rsqrt16(d):
    i = plsc.bitcast(d, jnp.int32)
    i = jnp.int32(0x5F3759DF) - lax.shift_right_logical(i, 1)
    y = plsc.bitcast(i, jnp.float32)
    for _ in range(3):
        y = y * (1.5 - 0.5 * d * y * y)
    return y


@functools.partial(
    pl.kernel,
    out_type=[
        jax.ShapeDtypeStruct((NC, N_PAD), jnp.float32),
        jax.ShapeDtypeStruct((N_PAD,), jnp.float32),
    ],
    mesh=_sc_mesh(),
    compiler_params=_SC_PARAMS_NL,
    scratch_types=[
        pltpu.VMEM((NPT, H), jnp.float32),
        pltpu.VMEM((NPT, H), jnp.float32),
        pltpu.VMEM((NPT, H), jnp.float32),
        pltpu.VMEM((NPT,), jnp.float32),
        pltpu.VMEM((NPT,), jnp.float32),
        pltpu.VMEM((NPT,), jnp.float32),
        pltpu.VMEM((NPT,), jnp.float32),
        pltpu.VMEM((16,), jnp.float32),
        pltpu.VMEM((16,), jnp.float32),
        pltpu.VMEM((2, CHUNK), jnp.int32),
        pltpu.VMEM((2, SUB, IB), jnp.int32),
        pltpu.VMEM((2, CHUNK), jnp.float32),
        pltpu.VMEM((N_PAD,), jnp.float32),
        pltpu.VMEM_SHARED((N_PAD,), jnp.float32),
        pltpu.VMEM_SHARED((N_PAD,), jnp.float32),
        pltpu.SemaphoreType.DMA,
        pltpu.SemaphoreType.DMA,
    ],
)
def _sc_layer2(agg1_hbm, h0p_hbm, indeg_hbm, b1_hbm, w2_hbm, srcf_hbm,
               dst_hbm, out_hbm, sp_hbm,
               a0_v, a1_v, h0_v, i0_v, i1_v, dinv_v, sps_v, b1_v, w2_v,
               src_v, dst_v, vals_v, spt_v, sp_sh, acc_sh, ssem0, ssem1):
    c = lax.axis_index("c")
    s = lax.axis_index("s")
    tile = c * NS + s
    ssems = (ssem0, ssem1)
    lo = s * NPT

    zeros16 = jnp.zeros((16,), jnp.float32)
    _fill(sps_v, NPT, zeros16)
    pltpu.sync_copy(sps_v, acc_sh.at[pl.ds(lo, NPT)])

    pltpu.sync_copy(agg1_hbm.at[0, pl.ds(lo, NPT)], a0_v)
    pltpu.sync_copy(agg1_hbm.at[1, pl.ds(lo, NPT)], a1_v)
    pltpu.sync_copy(h0p_hbm.at[pl.ds(lo, NPT)], h0_v)
    pltpu.sync_copy(indeg_hbm.at[0, pl.ds(lo, NPT)], i0_v)
    pltpu.sync_copy(indeg_hbm.at[1, pl.ds(lo, NPT)], i1_v)
    pltpu.sync_copy(b1_hbm, b1_v)
    pltpu.sync_copy(w2_hbm, w2_v)

    def dv(i, carry):
        d = i0_v[pl.ds(i * 16, 16)] + i1_v[pl.ds(i * 16, 16)] + 1.0
        dinv_v[pl.ds(i * 16, 16)] = _rsqrt16(d)
        return carry

    lax.fori_loop(0, NPT // 16, dv, 0)

    b1r = b1_v[...]
    w2r = w2_v[...]
    lastlane = lax.broadcasted_iota(jnp.int32, (16,), 0) == 15

    def node(n, carry):
        idxn = jnp.full((16,), n, jnp.int32)
        dvn = plsc.load_gather(dinv_v, [idxn])
        row = a0_v[n] + a1_v[n] + h0_v[n]
        h = jnp.maximum(row * dvn + b1r, 0.0)
        cs = plsc.cumsum(h * w2r)
        plsc.store_scatter(sps_v, [idxn], cs * dvn, mask=lastlane)
        return carry

    lax.fori_loop(0, NPT, node, 0)

    pltpu.sync_copy(sps_v, sp_sh.at[pl.ds(lo, NPT)])

    @pl.when(c == 0)
    def _():
        pltpu.sync_copy(sps_v, sp_hbm.at[pl.ds(lo, NPT)])

    plsc.subcore_barrier()
    pltpu.sync_copy(sp_sh, spt_v)

    pending = [None, None]
    for k in range(NCHUNK):
        b = k & 1
        if pending[b] is not None:
            for d in pending[b]:
                d.wait()
        base = tile * ET + k * CHUNK
        r = tile * IRT + k * SUB
        pltpu.sync_copy(srcf_hbm.at[pl.ds(base, CHUNK)], src_v.at[b])
        pltpu.sync_copy(dst_hbm.at[pl.ds(r, SUB)], dst_v.at[b])

        def gat(i, carry, b=b):
            idx16 = src_v[b, pl.ds(i * 16, 16)]
            vals_v[b, pl.ds(i * 16, 16)] = plsc.load_gather(spt_v, [idx16])
            return carry

        lax.fori_loop(0, CHUNK // 16, gat, 0)
        pending[b] = [
            pltpu.async_copy(vals_v.at[b, pl.ds(j * IB, IB)],
                             acc_sh.at[dst_v.at[b, j]], ssems[b], add=True)
            for j in range(SUB)
        ]
    for b in range(2):
        for d in pending[b]:
            d.wait()
    plsc.subcore_barrier()
    pltpu.sync_copy(acc_sh.at[pl.ds(lo, NPT)],
                    out_hbm.at[c, pl.ds(lo, NPT)])


def _tca_body(x_ref, w1_ref, h0_ref):
    h0_ref[...] = jnp.dot(x_ref[...], w1_ref[...],
                          preferred_element_type=jnp.float32)


_tca = pl.pallas_call(
    _tca_body,
    out_shape=jax.ShapeDtypeStruct((N_PAD, H), jnp.float32),
)


def _tcb_body(h0_ref, indeg_ref, h0p_ref, dinv_ref):
    ind = indeg_ref[...]
    deg = jnp.transpose(ind[0:1, :] + ind[1:2, :] + 1.0)
    dinv = lax.rsqrt(deg)
    h0p_ref[...] = h0_ref[...] * dinv
    dinv_ref[...] = dinv


_tcb = pl.pallas_call(
    _tcb_body,
    out_shape=[
        jax.ShapeDtypeStruct((N_PAD, H), jnp.float32),
        jax.ShapeDtypeStruct((N_PAD, 1), jnp.float32),
    ],
)


def _tc3_body(agg2_ref, sp_ref, dinv_ref, b2_ref, out_ref):
    a2 = agg2_ref[...]
    tcol = jnp.transpose(a2[0:1, :] + a2[1:2, :] + sp_ref[...])
    t = dinv_ref[...] * tcol + b2_ref[...]
    out_ref[...] = jax.nn.sigmoid(t)


_tc3 = pl.pallas_call(
    _tc3_body,
    out_shape=jax.ShapeDtypeStruct((N_PAD, 1), jnp.float32),
)


_PAD_IDX = np.asarray(
    N + (np.arange(E_PAD - 320000, dtype=np.int32) % (N_PAD - N)),
    dtype=np.int32)


def kernel(x, edge_index, W1, b1, W2, b2):
    x_pad = jnp.pad(x, ((0, N_PAD - N), (0, 0)))
    n_fake = E_PAD - edge_index.shape[1]
    padv = jnp.asarray(_PAD_IDX[:n_fake])
    srcf = jnp.concatenate([edge_index[0], padv])
    dstf = jnp.concatenate([edge_index[1], padv])
    src2d = srcf.reshape(EROWS, IB)
    dst2d = dstf.reshape(EROWS, IB)

    h0 = _tca(x_pad, W1)
    indeg = _sc_degree(dst2d)
    h0p, dinv = _tcb(h0, indeg)
    agg1 = _sc_rows(h0p, src2d, dst2d)
    agg2, sp = _sc_layer2(agg1, h0p, indeg, b1, W2.reshape(H), srcf, dst2d)
    out = _tc3(agg2, sp.reshape(1, N_PAD), dinv, b2.reshape(1, 1))
    return out[:N]
